# Initial kernel scaffold; baseline (speedup 1.0000x reference)
import jax
import jax.numpy as jnp
from jax.experimental import pallas as pl


def kernel(x, columns):
    raise NotImplementedError("write your pallas kernel here")


# speedup vs baseline: 2.0482x; 2.0482x over previous
import jax
import jax.numpy as jnp
from jax import lax
from jax.experimental import pallas as pl

N_ROWS = 16384
N_FEATS = 512
OUT_F = 64
BLOCK_ROWS = 1024


def _gather_body(cols_ref, x_ref, o_ref):
    cols = cols_ref[0:1, :]
    iota_c = lax.broadcasted_iota(jnp.int32, (N_FEATS, OUT_F), 0)
    onehot = (iota_c == cols).astype(jnp.float32)
    o_ref[...] = jnp.dot(x_ref[...], onehot, preferred_element_type=jnp.float32)


def kernel(x, columns):
    cols2d = jnp.broadcast_to(columns.astype(jnp.int32)[None, :], (8, OUT_F))
    out = pl.pallas_call(
        _gather_body,
        grid=(N_ROWS // BLOCK_ROWS,),
        in_specs=[
            pl.BlockSpec((8, OUT_F), lambda i: (0, 0)),
            pl.BlockSpec((BLOCK_ROWS, N_FEATS), lambda i: (i, 0)),
        ],
        out_specs=pl.BlockSpec((BLOCK_ROWS, OUT_F), lambda i: (i, 0)),
        out_shape=jax.ShapeDtypeStruct((N_ROWS, OUT_F), jnp.float32),
    )(cols2d, x)
    return out
